# baseline (speedup 1.0000x reference)
import functools

import jax
import jax.numpy as jnp
from jax import lax
from jax.experimental import pallas as pl
from jax.experimental.pallas import tpu as pltpu
from jax.experimental.pallas import tpu_sc as plsc

K = 16
S = 64


def _sc_phase1_body(N, D, NC, NS, vec_hbm, idx4_hbm, sums_hbm,
                    vbuf, abuf, zbuf, zidx, acc_sh):
    NW = NC * NS
    rows_per_w = N // NW
    cidx = lax.axis_index("c")
    sidx = lax.axis_index("s")
    wid = sidx * NC + cidx
    base = wid * rows_per_w
    zeros16 = jnp.zeros((16,), jnp.float32)
    iota = lax.iota(jnp.int32, 16)

    for m in range(8 * K):
        zbuf[m, pl.ds(0, 16)] = zeros16
        zbuf[m, pl.ds(16, 16)] = zeros16
    for c in range(8):
        zidx[pl.ds(16 * c, 16)] = iota + 16 * c

    @pl.when(sidx == 0)
    def _zero():
        pltpu.sync_copy(zbuf, acc_sh.at[zidx])

    plsc.subcore_barrier()

    def chunk(t, carry):
        cb = pl.multiple_of(base + t * S, S)
        pltpu.sync_copy(vec_hbm.at[pl.ds(cb, S)], vbuf.at[pl.ds(0, S)])
        pltpu.sync_copy(
            idx4_hbm.at[pl.ds(pl.multiple_of(cb * 4, 4 * S), 4 * S)], abuf)
        pltpu.sync_copy(vbuf, acc_sh.at[abuf], add=True)
        return carry

    lax.fori_loop(0, rows_per_w // S, chunk, 0)
    plsc.subcore_barrier()

    @pl.when(sidx == 0)
    def _out():
        pltpu.sync_copy(acc_sh.at[pl.ds(16, K)], sums_hbm.at[cidx])


def _counts_body(nb, assign_ref, cnt_ref, cnt_acc):
    i = pl.program_id(0)

    @pl.when(i == 0)
    def _init():
        cnt_acc[...] = jnp.zeros_like(cnt_acc)

    a = assign_ref[0]
    kio = lax.broadcasted_iota(jnp.int32, (K, a.shape[1]), 0)
    onehot = (a == kio).astype(jnp.float32)
    cnt_acc[...] += jnp.sum(onehot, axis=1, keepdims=True)

    @pl.when(i == nb - 1)
    def _fin():
        cnt_ref[...] = cnt_acc[...]


def _phase2_body(nc, sums_ref, counts_ref, vec_ref, out_ref, cent_out,
                 cent_s, c2_s):
    i = pl.program_id(0)

    @pl.when(i == 0)
    def _init():
        sums = sums_ref[0]
        for w in range(1, nc):
            sums = sums + sums_ref[w]
        cent = sums / counts_ref[...]
        cent_s[...] = cent
        c2_s[...] = jnp.sum(cent * cent, axis=1, keepdims=True)
        cent_out[...] = cent

    c = cent_s[...].astype(jnp.bfloat16)
    cross = lax.dot_general(
        c, vec_ref[...].astype(jnp.bfloat16), (((1,), (1,)), ((), ())),
        preferred_element_type=jnp.float32)
    score = c2_s[...] - 2.0 * cross
    min_v = jnp.min(score, axis=0, keepdims=True)
    kio = lax.broadcasted_iota(jnp.int32, score.shape, 0)
    idx = jnp.min(jnp.where(score == min_v, kio, K), axis=0, keepdims=True)
    out_ref[...] = idx[None]


def kernel(vectors, assignment):
    N, D = vectors.shape
    info = plsc.get_sparse_core_info()
    NC, NS = info.num_cores, info.num_subcores
    mesh = plsc.VectorSubcoreMesh(core_axis_name="c", subcore_axis_name="s")
    idx4 = jnp.repeat(assignment * 4 + 64, 4)

    sums = pl.kernel(
        functools.partial(_sc_phase1_body, N, D, NC, NS),
        mesh=mesh,
        out_type=jax.ShapeDtypeStruct((NC, K, D), jnp.float32),
        scratch_types=[
            pltpu.VMEM((4 * S, D), jnp.float32),
            pltpu.VMEM((4 * S,), jnp.int32),
            pltpu.VMEM((8 * K, D), jnp.float32),
            pltpu.VMEM((8 * K,), jnp.int32),
            pltpu.VMEM_SHARED((2 * K, D), jnp.float32),
        ],
    )(vectors, idx4)

    B = 16384
    nb = N // B
    assign3 = assignment.reshape(nb, 1, B)

    counts = pl.pallas_call(
        functools.partial(_counts_body, nb),
        grid=(nb,),
        in_specs=[pl.BlockSpec((1, 1, B), lambda i: (i, 0, 0))],
        out_specs=pl.BlockSpec((K, 1), lambda i: (0, 0)),
        out_shape=jax.ShapeDtypeStruct((K, 1), jnp.float32),
        scratch_shapes=[pltpu.VMEM((K, 1), jnp.float32)],
    )(assign3)

    new_assign3, centroids = pl.pallas_call(
        functools.partial(_phase2_body, NC),
        grid=(nb,),
        in_specs=[
            pl.BlockSpec((NC, K, D), lambda i: (0, 0, 0)),
            pl.BlockSpec((K, 1), lambda i: (0, 0)),
            pl.BlockSpec((B, D), lambda i: (i, 0)),
        ],
        out_specs=[
            pl.BlockSpec((1, 1, B), lambda i: (i, 0, 0)),
            pl.BlockSpec((K, D), lambda i: (0, 0)),
        ],
        out_shape=[
            jax.ShapeDtypeStruct((nb, 1, B), jnp.int32),
            jax.ShapeDtypeStruct((K, D), jnp.float32),
        ],
        scratch_shapes=[
            pltpu.VMEM((K, D), jnp.float32),
            pltpu.VMEM((K, 1), jnp.float32),
        ],
    )(sums, counts, vectors)

    return centroids, new_assign3.reshape(N)
